# Initial kernel scaffold; baseline (speedup 1.0000x reference)
import jax
import jax.numpy as jnp
from jax.experimental import pallas as pl


def kernel(attention):
    raise NotImplementedError("write your pallas kernel here")


# speedup vs baseline: 20.2173x; 20.2173x over previous
import functools

import jax
import jax.numpy as jnp
from jax.experimental import pallas as pl
from jax.experimental.pallas import tpu as pltpu

_K_BY_HEAD = (10, 20, 40, 500)
_ROWS_PER_BLOCK = 256


def _monotone_i32(x):
    b = jax.lax.bitcast_convert_type(x, jnp.int32)
    return jnp.where(b >= 0, b, b ^ jnp.int32(0x7FFFFFFF))


def _topk_softmax_block(k_ref, x_ref, o_ref):
    x = x_ref[0]
    m = _monotone_i32(x)
    k = k_ref[pl.program_id(0)]

    lo0 = jnp.min(m, axis=-1, keepdims=True)
    hi0 = jnp.max(m, axis=-1, keepdims=True)

    def body(_, state):
        lo, hi = state
        floor_mid = (lo >> 1) + (hi >> 1) + (lo & hi & 1)
        mid = floor_mid + ((lo ^ hi) & 1)
        cnt = jnp.sum((m >= mid).astype(jnp.int32), axis=-1, keepdims=True)
        ge = cnt >= k
        lo = jnp.where(ge, mid, lo)
        hi = jnp.where(ge, hi, mid - 1)
        return lo, hi

    t, _ = jax.lax.fori_loop(0, 32, body, (lo0, hi0), unroll=False)

    keep = m >= t
    rowmax = jnp.max(x, axis=-1, keepdims=True)
    e = jnp.where(keep, jnp.exp(x - rowmax), 0.0)
    s = jnp.sum(e, axis=-1, keepdims=True)
    o_ref[0] = e / s


@jax.jit
def kernel(attention):
    B, H, M, N = attention.shape
    S = B * H
    x = attention.reshape(S, M, N)
    ks = jnp.tile(
        jnp.array([min(k, N) for k in _K_BY_HEAD], dtype=jnp.int32), B
    )
    R = min(_ROWS_PER_BLOCK, M)
    nb = M // R

    grid_spec = pltpu.PrefetchScalarGridSpec(
        num_scalar_prefetch=1,
        grid=(S, nb),
        in_specs=[
            pl.BlockSpec((1, R, N), lambda s, j, k_ref: (s, j, 0)),
        ],
        out_specs=pl.BlockSpec((1, R, N), lambda s, j, k_ref: (s, j, 0)),
    )
    out = pl.pallas_call(
        _topk_softmax_block,
        grid_spec=grid_spec,
        out_shape=jax.ShapeDtypeStruct((S, M, N), jnp.float32),
        compiler_params=pltpu.CompilerParams(
            dimension_semantics=("parallel", "parallel"),
        ),
    )(ks, x)
    return out.reshape(B, H, M, N)
